# baseline (speedup 1.0000x reference)
import functools

import jax
import jax.numpy as jnp
from jax import lax
from jax.experimental import pallas as pl
from jax.experimental.pallas import tpu as pltpu
from jax.experimental.pallas import tpu_sc as plsc

HIDDEN_DIM = 64
BETA = 5.0
LANES = 16
NUM_WORKERS = 32
BSLAB = 128
NGROUPS = BSLAB // LANES
DBLK = HIDDEN_DIM // 8


def _sc_boost(emb4, ids4, table, weights, *, num_l):
    mesh = plsc.VectorSubcoreMesh(core_axis_name="c", subcore_axis_name="s")

    @functools.partial(
        pl.kernel,
        out_type=jax.ShapeDtypeStruct(emb4.shape, jnp.float32),
        mesh=mesh,
        compiler_params=pltpu.CompilerParams(
            needs_layout_passes=False, use_tc_tiling_on_sc=False),
        scratch_types=[
            pltpu.VMEM((num_l // 8, 8, BSLAB), jnp.int32),
            pltpu.VMEM((BSLAB,), jnp.float32),
            pltpu.VMEM((BSLAB,), jnp.float32),
            pltpu.VMEM((BSLAB, HIDDEN_DIM), jnp.float32),
            pltpu.VMEM((BSLAB, HIDDEN_DIM), jnp.float32),
            pltpu.VMEM((DBLK, 8, BSLAB), jnp.float32),
            pltpu.VMEM((DBLK, 8, BSLAB), jnp.float32),
            pltpu.SemaphoreType.DMA,
            pltpu.SemaphoreType.DMA,
            pltpu.SemaphoreType.DMA,
            pltpu.SemaphoreType.DMA,
        ],
    )
    def k(emb_hbm, ids_hbm, tab_hbm, w_hbm, out_hbm,
          ids_v, w0, w1, rows0, rows1, eb0, eb1,
          sem_in0, sem_in1, sem_out0, sem_out1):
        num_cores = jax.lax.axis_size("c")
        wid = lax.axis_index("s") * num_cores + lax.axis_index("c")
        pltpu.sync_copy(ids_hbm.at[:, wid, :, :], ids_v)

        bufs = ((w0, rows0, eb0, sem_in0, sem_out0),
                (w1, rows1, eb1, sem_in1, sem_out1))

        def idx_ref(step):
            return ids_v.at[lax.div(step, 8), lax.rem(step, 8)]

        def issue_in(b, step):
            w_v, rows_v, eb_v, sem_in, _ = bufs[b]
            idx = idx_ref(step)
            pltpu.async_copy(tab_hbm.at[idx], rows_v, sem_in)
            pltpu.async_copy(w_hbm.at[idx], w_v, sem_in)
            pltpu.async_copy(
                emb_hbm.at[pl.ds(step * DBLK, DBLK), wid, :, :], eb_v, sem_in)

        def wait_in(b, step):
            w_v, rows_v, eb_v, sem_in, _ = bufs[b]
            idx = idx_ref(step)
            pltpu.make_async_copy(tab_hbm.at[idx], rows_v, sem_in).wait()
            pltpu.make_async_copy(w_hbm.at[idx], w_v, sem_in).wait()
            pltpu.make_async_copy(
                emb_hbm.at[pl.ds(step * DBLK, DBLK), wid, :, :], eb_v,
                sem_in).wait()

        def issue_out(b, step):
            eb_v, sem_out = bufs[b][2], bufs[b][4]
            pltpu.async_copy(
                eb_v, out_hbm.at[pl.ds(step * DBLK, DBLK), wid, :, :], sem_out)

        def wait_out(b, step):
            eb_v, sem_out = bufs[b][2], bufs[b][4]
            pltpu.make_async_copy(
                eb_v, out_hbm.at[pl.ds(step * DBLK, DBLK), wid, :, :],
                sem_out).wait()

        def compute(b, step):
            w_v, rows_v, eb_v, _, _ = bufs[b]
            iota = lax.iota(jnp.int32, LANES)
            rowi = [iota + (jg * LANES) for jg in range(NGROUPS)]
            sj = [w_v[pl.ds(jg * LANES, LANES)] * BETA for jg in range(NGROUPS)]

            def dblock(dblk, _):
                for din in range(8):
                    colv = jnp.full((LANES,), dblk * 8 + din, jnp.int32)
                    for jg in range(NGROUPS):
                        sl = pl.ds(jg * LANES, LANES)
                        val = plsc.load_gather(rows_v, [rowi[jg], colv])
                        eb_v[dblk, din, sl] = eb_v[dblk, din, sl] + sj[jg] * val
                return 0

            lax.fori_loop(0, DBLK, dblock, 0)

        issue_in(0, 0)

        def pair(ii, _):
            for b in range(2):
                step = 2 * ii + b
                o = 1 - b

                @pl.when(step + 1 < num_l)
                def _():
                    @pl.when(step >= 1)
                    def _():
                        wait_out(o, step)
                    issue_in(o, step + 1)

                wait_in(b, step)
                compute(b, step)
                issue_out(b, step)
            return 0

        lax.fori_loop(0, num_l // 2, pair, 0)
        wait_out(0, num_l - 2)
        wait_out(1, num_l - 1)

    return k(emb4, ids4, table, weights)


def kernel(embeddings, token_ids, agency_matrix, boosting_weights):
    b, l, d = embeddings.shape
    emb4 = jnp.transpose(embeddings, (1, 2, 0)).reshape(
        l, d // 8, 8, NUM_WORKERS, BSLAB).transpose(0, 1, 3, 2, 4).reshape(
        l * (d // 8), NUM_WORKERS, 8, BSLAB)
    ids4 = jnp.transpose(token_ids, (1, 0)).astype(jnp.int32).reshape(
        l // 8, 8, NUM_WORKERS, BSLAB).transpose(0, 2, 1, 3)
    out4 = _sc_boost(emb4, ids4, agency_matrix, boosting_weights, num_l=l)
    out_t = out4.reshape(l, d // 8, NUM_WORKERS, 8, BSLAB).transpose(
        0, 1, 3, 2, 4).reshape(l, d, b)
    return jnp.transpose(out_t, (2, 0, 1))


# speedup vs baseline: 1.5753x; 1.5753x over previous
import functools

import jax
import jax.numpy as jnp
from jax import lax
from jax.experimental import pallas as pl
from jax.experimental.pallas import tpu as pltpu
from jax.experimental.pallas import tpu_sc as plsc

HIDDEN_DIM = 64
BETA = 5.0
LANES = 16
NUM_WORKERS = 32
BSLAB = 128
NGROUPS = BSLAB // LANES
DBLK = HIDDEN_DIM // 8
RES_PITCH = HIDDEN_DIM + 1


def _sc_boost(emb4, ids4, table, weights, *, num_l):
    mesh = plsc.VectorSubcoreMesh(core_axis_name="c", subcore_axis_name="s")

    @functools.partial(
        pl.kernel,
        out_type=jax.ShapeDtypeStruct(emb4.shape, jnp.float32),
        mesh=mesh,
        compiler_params=pltpu.CompilerParams(
            needs_layout_passes=False, use_tc_tiling_on_sc=False),
        scratch_types=[
            pltpu.VMEM((num_l // 8, 8, BSLAB), jnp.int32),
            pltpu.VMEM((BSLAB,), jnp.float32),
            pltpu.VMEM((BSLAB,), jnp.float32),
            pltpu.VMEM((BSLAB, HIDDEN_DIM), jnp.float32),
            pltpu.VMEM((BSLAB, HIDDEN_DIM), jnp.float32),
            pltpu.VMEM((DBLK, 8, BSLAB), jnp.float32),
            pltpu.VMEM((DBLK, 8, BSLAB), jnp.float32),
            pltpu.VMEM((BSLAB, RES_PITCH), jnp.float32),
            pltpu.SemaphoreType.DMA,
            pltpu.SemaphoreType.DMA,
            pltpu.SemaphoreType.DMA,
            pltpu.SemaphoreType.DMA,
        ],
    )
    def k(emb_hbm, ids_hbm, tab_hbm, w_hbm, out_hbm,
          ids_v, w0, w1, rows0, rows1, eb0, eb1, res_v,
          sem_in0, sem_in1, sem_out0, sem_out1):
        num_cores = jax.lax.axis_size("c")
        wid = lax.axis_index("s") * num_cores + lax.axis_index("c")
        pltpu.sync_copy(ids_hbm.at[:, wid, :, :], ids_v)

        bufs = ((w0, rows0, eb0, sem_in0, sem_out0),
                (w1, rows1, eb1, sem_in1, sem_out1))

        def idx_ref(step):
            return ids_v.at[lax.div(step, 8), lax.rem(step, 8)]

        def issue_in(b, step):
            w_v, rows_v, eb_v, sem_in, _ = bufs[b]
            idx = idx_ref(step)
            pltpu.async_copy(tab_hbm.at[idx], rows_v, sem_in)
            pltpu.async_copy(w_hbm.at[idx], w_v, sem_in)
            pltpu.async_copy(
                emb_hbm.at[pl.ds(step * DBLK, DBLK), wid, :, :], eb_v, sem_in)

        def wait_in(b, step):
            w_v, rows_v, eb_v, sem_in, _ = bufs[b]
            idx = idx_ref(step)
            pltpu.make_async_copy(tab_hbm.at[idx], rows_v, sem_in).wait()
            pltpu.make_async_copy(w_hbm.at[idx], w_v, sem_in).wait()
            pltpu.make_async_copy(
                emb_hbm.at[pl.ds(step * DBLK, DBLK), wid, :, :], eb_v,
                sem_in).wait()

        def issue_out(b, step):
            eb_v, sem_out = bufs[b][2], bufs[b][4]
            pltpu.async_copy(
                eb_v, out_hbm.at[pl.ds(step * DBLK, DBLK), wid, :, :], sem_out)

        def wait_out(b, step):
            eb_v, sem_out = bufs[b][2], bufs[b][4]
            pltpu.make_async_copy(
                eb_v, out_hbm.at[pl.ds(step * DBLK, DBLK), wid, :, :],
                sem_out).wait()

        def compute(b, step):
            w_v, rows_v, eb_v, _, _ = bufs[b]

            def stage4(q, _):
                for tt in range(4):
                    t = q * 4 + tt
                    for j in range(HIDDEN_DIM // LANES):
                        sl = pl.ds(j * LANES, LANES)
                        res_v[t, sl] = rows_v[t, sl]
                return 0

            lax.fori_loop(0, BSLAB // 4, stage4, 0)

            iota = lax.iota(jnp.int32, LANES)
            rowi = [iota + (jg * LANES) for jg in range(NGROUPS)]
            sj = [w_v[pl.ds(jg * LANES, LANES)] * BETA for jg in range(NGROUPS)]

            def dblock(dblk, _):
                for din in range(8):
                    colv = jnp.full((LANES,), dblk * 8 + din, jnp.int32)
                    vals = [plsc.load_gather(res_v, [rowi[jg], colv])
                            for jg in range(NGROUPS)]
                    embs = [eb_v[dblk, din, pl.ds(jg * LANES, LANES)]
                            for jg in range(NGROUPS)]
                    for jg in range(NGROUPS):
                        eb_v[dblk, din, pl.ds(jg * LANES, LANES)] = (
                            embs[jg] + sj[jg] * vals[jg])
                return 0

            lax.fori_loop(0, DBLK, dblock, 0)

        issue_in(0, 0)

        def pair(ii, _):
            for b in range(2):
                step = 2 * ii + b
                o = 1 - b

                @pl.when(step + 1 < num_l)
                def _():
                    @pl.when(step >= 1)
                    def _():
                        wait_out(o, step)
                    issue_in(o, step + 1)

                wait_in(b, step)
                compute(b, step)
                issue_out(b, step)
            return 0

        lax.fori_loop(0, num_l // 2, pair, 0)
        wait_out(0, num_l - 2)
        wait_out(1, num_l - 1)

    return k(emb4, ids4, table, weights)


def kernel(embeddings, token_ids, agency_matrix, boosting_weights):
    b, l, d = embeddings.shape
    emb4 = jnp.transpose(embeddings, (1, 2, 0)).reshape(
        l, d // 8, 8, NUM_WORKERS, BSLAB).transpose(0, 1, 3, 2, 4).reshape(
        l * (d // 8), NUM_WORKERS, 8, BSLAB)
    ids4 = jnp.transpose(token_ids, (1, 0)).astype(jnp.int32).reshape(
        l // 8, 8, NUM_WORKERS, BSLAB).transpose(0, 2, 1, 3)
    out4 = _sc_boost(emb4, ids4, agency_matrix, boosting_weights, num_l=l)
    out_t = out4.reshape(l, d // 8, NUM_WORKERS, 8, BSLAB).transpose(
        0, 1, 3, 2, 4).reshape(l, d, b)
    return jnp.transpose(out_t, (2, 0, 1))


# speedup vs baseline: 1.9732x; 1.2525x over previous
import functools

import jax
import jax.numpy as jnp
from jax import lax
from jax.experimental import pallas as pl
from jax.experimental.pallas import tpu as pltpu
from jax.experimental.pallas import tpu_sc as plsc

HIDDEN_DIM = 64
BETA = 5.0
LANES = 16
NUM_WORKERS = 32
BSLAB = 128
NGROUPS = BSLAB // LANES
DBLK = HIDDEN_DIM // 8
RES_PITCH = HIDDEN_DIM + 1


def _sc_boost(emb4, ids4, table, weights, *, num_l):
    assert num_l % 4 == 0
    mesh = plsc.VectorSubcoreMesh(core_axis_name="c", subcore_axis_name="s")

    @functools.partial(
        pl.kernel,
        out_type=jax.ShapeDtypeStruct(emb4.shape, jnp.float32),
        mesh=mesh,
        compiler_params=pltpu.CompilerParams(
            needs_layout_passes=False, use_tc_tiling_on_sc=False),
        scratch_types=[
            pltpu.VMEM((num_l // 8, 8, BSLAB), jnp.int32),
            [pltpu.VMEM((BSLAB,), jnp.float32)] * 4,
            [pltpu.VMEM((BSLAB, HIDDEN_DIM), jnp.float32)] * 4,
            [pltpu.VMEM((DBLK, 8, BSLAB), jnp.float32)] * 4,
            pltpu.VMEM((BSLAB * RES_PITCH,), jnp.float32),
            [pltpu.SemaphoreType.DMA] * 4,
            [pltpu.SemaphoreType.DMA] * 4,
        ],
    )
    def k(emb_hbm, ids_hbm, tab_hbm, w_hbm, out_hbm,
          ids_v, w_b, rows_b, eb_b, res_v, sem_in, sem_out):
        num_cores = jax.lax.axis_size("c")
        wid = lax.axis_index("s") * num_cores + lax.axis_index("c")
        pltpu.sync_copy(ids_hbm.at[:, wid, :, :], ids_v)

        def idx_ref(step):
            return ids_v.at[lax.div(step, 8), lax.rem(step, 8)]

        def issue_in(b, step):
            idx = idx_ref(step)
            pltpu.async_copy(tab_hbm.at[idx], rows_b[b], sem_in[b])
            pltpu.async_copy(w_hbm.at[idx], w_b[b], sem_in[b])
            pltpu.async_copy(
                emb_hbm.at[pl.ds(step * DBLK, DBLK), wid, :, :], eb_b[b],
                sem_in[b])

        def wait_in(b, step):
            idx = idx_ref(step)
            pltpu.make_async_copy(tab_hbm.at[idx], rows_b[b], sem_in[b]).wait()
            pltpu.make_async_copy(w_hbm.at[idx], w_b[b], sem_in[b]).wait()
            pltpu.make_async_copy(
                emb_hbm.at[pl.ds(step * DBLK, DBLK), wid, :, :], eb_b[b],
                sem_in[b]).wait()

        def issue_out(b, step):
            pltpu.async_copy(
                eb_b[b], out_hbm.at[pl.ds(step * DBLK, DBLK), wid, :, :],
                sem_out[b])

        def wait_out(b, step):
            pltpu.make_async_copy(
                eb_b[b], out_hbm.at[pl.ds(step * DBLK, DBLK), wid, :, :],
                sem_out[b]).wait()

        def compute(b, step):
            w_v, rows_v, eb_v = w_b[b], rows_b[b], eb_b[b]

            def stage4(q, _):
                for tt in range(4):
                    t = q * 4 + tt
                    for j in range(HIDDEN_DIM // LANES):
                        res_v[pl.ds(t * RES_PITCH + j * LANES, LANES)] = (
                            rows_v[t, pl.ds(j * LANES, LANES)])
                return 0

            lax.fori_loop(0, BSLAB // 4, stage4, 0)

            iota = lax.iota(jnp.int32, LANES)
            fb = [(iota + jg * LANES) * RES_PITCH for jg in range(NGROUPS)]
            sj = [w_v[pl.ds(jg * LANES, LANES)] * BETA for jg in range(NGROUPS)]

            def dblock(dblk, _):
                for din in range(8):
                    colv = jnp.full((LANES,), dblk * 8 + din, jnp.int32)
                    vals = [plsc.load_gather(res_v, [fb[jg] + colv])
                            for jg in range(NGROUPS)]
                    for jg in range(NGROUPS):
                        plsc.addupdate(
                            eb_v.at[dblk, din, pl.ds(jg * LANES, LANES)],
                            sj[jg] * vals[jg])
                return 0

            lax.fori_loop(0, DBLK, dblock, 0)

        issue_in(0, 0)
        issue_in(1, 1)

        def quad(ii, _):
            for qb in range(4):
                step = 4 * ii + qb
                nxt = step + 2
                s2 = (qb + 2) % 4

                @pl.when(nxt < num_l)
                def _():
                    @pl.when(step >= 2)
                    def _():
                        wait_out(s2, step - 2)
                    issue_in(s2, nxt)

                wait_in(qb, step)
                compute(qb, step)
                issue_out(qb, step)
            return 0

        lax.fori_loop(0, num_l // 4, quad, 0)
        for qb in range(4):
            wait_out(qb, num_l - 4 + qb)

    return k(emb4, ids4, table, weights)


def kernel(embeddings, token_ids, agency_matrix, boosting_weights):
    b, l, d = embeddings.shape
    emb4 = jnp.transpose(embeddings, (1, 2, 0)).reshape(
        l, d // 8, 8, NUM_WORKERS, BSLAB).transpose(0, 1, 3, 2, 4).reshape(
        l * (d // 8), NUM_WORKERS, 8, BSLAB)
    ids4 = jnp.transpose(token_ids, (1, 0)).astype(jnp.int32).reshape(
        l // 8, 8, NUM_WORKERS, BSLAB).transpose(0, 2, 1, 3)
    out4 = _sc_boost(emb4, ids4, agency_matrix, boosting_weights, num_l=l)
    out_t = out4.reshape(l, d // 8, NUM_WORKERS, 8, BSLAB).transpose(
        0, 1, 3, 2, 4).reshape(l, d, b)
    return jnp.transpose(out_t, (2, 0, 1))
